# baseline (speedup 1.0000x reference)
import functools

import jax
import jax.numpy as jnp
from jax import lax
from jax.experimental import pallas as pl
from jax.experimental.pallas import tpu as pltpu
from jax.experimental.pallas import tpu_sc as plsc

N = 3_200_000
D = 16
S = 100_000
ROWS = N // 128
S_PAD = 102_400
NW = 32
SEG_PER_W = S_PAD // NW
SEG_PER_SUB = S_PAD // 16
ROWS_PER_W = 784
KR = 8

_f32 = jnp.float32


HS_ROWS = N * D // 128
TC_GRID = 125
HS_BLK = HS_ROWS // TC_GRID
OUT_BLK = ROWS // TC_GRID


def _tc_body(ht_ref, hp_ref, pt_ref, pp_ref, rl_ref, pk_ref):
    d = hp_ref[...] - ht_ref[...]
    sq = d * d
    t = jnp.sum(sq.reshape(HS_BLK, 8, D), axis=-1)
    rl_ref[...] = t.reshape(OUT_BLK, 128) * (1.0 / D)
    e = pp_ref[...] - pt_ref[...]
    pk_ref[...] = e * e


def _tc_row_losses(hs_t2, hs_p2, pks_t2, pks_p2):
    rl_hs, l_pks = pl.pallas_call(
        _tc_body,
        grid=(TC_GRID,),
        in_specs=[
            pl.BlockSpec((HS_BLK, 128), lambda i: (i, 0)),
            pl.BlockSpec((HS_BLK, 128), lambda i: (i, 0)),
            pl.BlockSpec((OUT_BLK, 128), lambda i: (i, 0)),
            pl.BlockSpec((OUT_BLK, 128), lambda i: (i, 0)),
        ],
        out_specs=[
            pl.BlockSpec((OUT_BLK, 128), lambda i: (i, 0)),
            pl.BlockSpec((OUT_BLK, 128), lambda i: (i, 0)),
        ],
        out_shape=[
            jax.ShapeDtypeStruct((ROWS, 128), _f32),
            jax.ShapeDtypeStruct((ROWS, 128), _f32),
        ],
    )(hs_t2, hs_p2, pks_t2, pks_p2)
    return rl_hs, l_pks


_mesh = plsc.VectorSubcoreMesh(core_axis_name="c", subcore_axis_name="s")


@functools.partial(
    pl.kernel,
    out_type=jax.ShapeDtypeStruct((2, 4, S_PAD), _f32),
    mesh=_mesh,
    scratch_types=[
        pltpu.VMEM_SHARED((S_PAD,), _f32),
        pltpu.VMEM_SHARED((S_PAD,), _f32),
        pltpu.VMEM_SHARED((S_PAD,), _f32),
        pltpu.VMEM_SHARED((S_PAD,), _f32),
        pltpu.VMEM((KR, 128), _f32),
        pltpu.VMEM((KR * 128,), jnp.int32),
        pltpu.VMEM((128,), _f32),
        pltpu.SemaphoreType.DMA,
    ],
)
def _sc_accum(rl_hbm, idhs_hbm, lpks_hbm, idpks_hbm, zeros_hbm, out_hbm,
              tsum_hs, tcnt_hs, tsum_pks, tcnt_pks, vals_v, ids_v, ones_v,
              sem):
    c = lax.axis_index("c")
    s = lax.axis_index("s")
    wid = c * 16 + s

    for tab in (tsum_hs, tcnt_hs, tsum_pks, tcnt_pks):
        pltpu.sync_copy(zeros_hbm, tab.at[pl.ds(s * SEG_PER_SUB, SEG_PER_SUB)])
    for g in range(8):
        ones_v[pl.ds(g * 16, 16)] = jnp.ones((16,), _f32)
    plsc.subcore_barrier()

    row_lo = wid * ROWS_PER_W
    nrows = jnp.minimum(ROWS_PER_W, ROWS - row_lo)
    nchunks = nrows // KR

    def scatter_loop(vhbm, ihbm, tsum, tcnt):
        def chunk(ci, carry):
            base = row_lo + ci * KR
            pltpu.sync_copy(vhbm.at[pl.ds(base, KR)], vals_v)
            pltpu.sync_copy(ihbm.at[pl.ds(base * 128, KR * 128)], ids_v)
            handles = []
            for j in range(KR):
                idv = ids_v.at[pl.ds(j * 128, 128)]
                handles.append(pltpu.async_copy(
                    vals_v.at[j], tsum.at[idv], sem, add=True))
                handles.append(pltpu.async_copy(
                    ones_v, tcnt.at[idv], sem, add=True))
            for h in handles:
                h.wait()
            return carry
        lax.fori_loop(0, nchunks, chunk, 0)

    scatter_loop(rl_hbm, idhs_hbm, tsum_hs, tcnt_hs)
    scatter_loop(lpks_hbm, idpks_hbm, tsum_pks, tcnt_pks)

    plsc.subcore_barrier()
    for t, tab in enumerate((tsum_hs, tcnt_hs, tsum_pks, tcnt_pks)):
        sl = pl.ds(s * SEG_PER_SUB, SEG_PER_SUB)
        pltpu.sync_copy(tab.at[sl], out_hbm.at[c, t, sl])


@functools.partial(
    pl.kernel,
    out_type=jax.ShapeDtypeStruct((NW, 48), _f32),
    mesh=_mesh,
    scratch_types=[
        [pltpu.VMEM((SEG_PER_W,), _f32) for _ in range(8)],
        pltpu.VMEM((SEG_PER_W,), _f32),
        pltpu.VMEM((SEG_PER_W,), _f32),
        pltpu.VMEM((48,), _f32),
    ],
)
def _sc_finalize(tables_hbm, pit_hbm, pip_hbm, out_hbm, tb, pit_v, pip_v,
                 obuf):
    c = lax.axis_index("c")
    s = lax.axis_index("s")
    wid = c * 16 + s
    base = wid * SEG_PER_W
    sl = pl.ds(base, SEG_PER_W)
    k = 0
    for t in range(4):
        for core in range(2):
            pltpu.sync_copy(tables_hbm.at[core, t, sl], tb[k])
            k += 1
    pltpu.sync_copy(pit_hbm.at[sl], pit_v)
    pltpu.sync_copy(pip_hbm.at[sl], pip_v)

    zero = jnp.zeros((16,), _f32)

    def body(i, accs):
        a_hs, a_pks, a_pi = accs
        v = pl.ds(i * 16, 16)
        sh = tb[0][v] + tb[1][v]
        ch = jnp.maximum(tb[2][v] + tb[3][v], 1.0)
        a_hs = a_hs + sh / ch
        sp = tb[4][v] + tb[5][v]
        cp = jnp.maximum(tb[6][v] + tb[7][v], 1.0)
        a_pks = a_pks + sp / cp
        dpi = pip_v[v] - pit_v[v]
        a_pi = a_pi + dpi * dpi
        return (a_hs, a_pks, a_pi)

    a_hs, a_pks, a_pi = lax.fori_loop(0, SEG_PER_W // 16, body,
                                      (zero, zero, zero))
    obuf[pl.ds(0, 16)] = a_hs
    obuf[pl.ds(16, 16)] = a_pks
    obuf[pl.ds(32, 16)] = a_pi
    pltpu.sync_copy(obuf, out_hbm.at[wid])


def kernel(y_hs_true, y_hs_pred, y_hs_batch, y_pks_true, y_pks_pred,
           y_pks_batch, y_pI_true, y_pI_pred):
    pks_t2 = y_pks_true.reshape(ROWS, 128)
    pks_p2 = y_pks_pred.reshape(ROWS, 128)
    hs_t2 = y_hs_true.reshape(HS_ROWS, 128)
    hs_p2 = y_hs_pred.reshape(HS_ROWS, 128)
    rl_hs, l_pks = _tc_row_losses(hs_t2, hs_p2, pks_t2, pks_p2)

    idhs = y_hs_batch.astype(jnp.int32)
    idpks = y_pks_batch.astype(jnp.int32)
    zeros_hbm = jnp.zeros((SEG_PER_SUB,), _f32)

    tables = _sc_accum(rl_hs, idhs, l_pks, idpks, zeros_hbm)

    pit = jnp.pad(y_pI_true, (0, S_PAD - S))
    pip = jnp.pad(y_pI_pred, (0, S_PAD - S))
    parts = _sc_finalize(tables, pit, pip)

    tot = jnp.sum(parts, axis=0)
    multitask = jnp.stack([
        jnp.sum(tot[0:16]), jnp.sum(tot[16:32]), jnp.sum(tot[32:48])
    ]) * (1.0 / S)
    return (multitask, jnp.zeros(1), jnp.zeros(1))


# speedup vs baseline: 1.2305x; 1.2305x over previous
import functools

import jax
import jax.numpy as jnp
from jax import lax
from jax.experimental import pallas as pl
from jax.experimental.pallas import tpu as pltpu
from jax.experimental.pallas import tpu_sc as plsc

N = 3_200_000
D = 16
S = 100_000
ROWS = N // 128
S_PAD = 102_400
NW = 32
SEG_PER_W = S_PAD // NW
SEG_PER_SUB = S_PAD // 16
ROWS_PER_W = 784
KR = 8

_f32 = jnp.float32


HS_ROWS = N * D // 128
TC_GRID = 125
HS_BLK = HS_ROWS // TC_GRID
OUT_BLK = ROWS // TC_GRID


def _tc_body(ht_ref, hp_ref, pt_ref, pp_ref, rl_ref, pk_ref):
    d = hp_ref[...] - ht_ref[...]
    sq = d * d
    c_idx = lax.broadcasted_iota(jnp.int32, (128, 128), 0)
    j_idx = lax.broadcasted_iota(jnp.int32, (128, 128), 1)
    g_mat = jnp.where(c_idx // D == j_idx % 8, 1.0 / D, 0.0).astype(_f32)
    t2 = jnp.dot(sq, g_mat, preferred_element_type=_f32)
    r_idx = lax.broadcasted_iota(jnp.int32, (16, 128), 0)
    l_idx = lax.broadcasted_iota(jnp.int32, (16, 128), 1)
    d_mask = (r_idx == l_idx // 8).astype(_f32)
    t3 = t2.reshape(OUT_BLK, 16, 128) * d_mask[None, :, :]
    rl_ref[...] = jnp.sum(t3, axis=1)
    e = pp_ref[...] - pt_ref[...]
    pk_ref[...] = e * e


def _tc_row_losses(hs_t2, hs_p2, pks_t2, pks_p2):
    rl_hs, l_pks = pl.pallas_call(
        _tc_body,
        grid=(TC_GRID,),
        in_specs=[
            pl.BlockSpec((HS_BLK, 128), lambda i: (i, 0)),
            pl.BlockSpec((HS_BLK, 128), lambda i: (i, 0)),
            pl.BlockSpec((OUT_BLK, 128), lambda i: (i, 0)),
            pl.BlockSpec((OUT_BLK, 128), lambda i: (i, 0)),
        ],
        out_specs=[
            pl.BlockSpec((OUT_BLK, 128), lambda i: (i, 0)),
            pl.BlockSpec((OUT_BLK, 128), lambda i: (i, 0)),
        ],
        out_shape=[
            jax.ShapeDtypeStruct((ROWS, 128), _f32),
            jax.ShapeDtypeStruct((ROWS, 128), _f32),
        ],
    )(hs_t2, hs_p2, pks_t2, pks_p2)
    return rl_hs, l_pks


_mesh = plsc.VectorSubcoreMesh(core_axis_name="c", subcore_axis_name="s")


@functools.partial(
    pl.kernel,
    out_type=jax.ShapeDtypeStruct((2, 4, S_PAD), _f32),
    mesh=_mesh,
    scratch_types=[
        pltpu.VMEM_SHARED((S_PAD,), _f32),
        pltpu.VMEM_SHARED((S_PAD,), _f32),
        pltpu.VMEM_SHARED((S_PAD,), _f32),
        pltpu.VMEM_SHARED((S_PAD,), _f32),
        pltpu.VMEM((KR, 128), _f32),
        pltpu.VMEM((KR * 128,), jnp.int32),
        pltpu.VMEM((128,), _f32),
        pltpu.SemaphoreType.DMA,
    ],
)
def _sc_accum(rl_hbm, idhs_hbm, lpks_hbm, idpks_hbm, zeros_hbm, out_hbm,
              tsum_hs, tcnt_hs, tsum_pks, tcnt_pks, vals_v, ids_v, ones_v,
              sem):
    c = lax.axis_index("c")
    s = lax.axis_index("s")
    wid = c * 16 + s

    for tab in (tsum_hs, tcnt_hs, tsum_pks, tcnt_pks):
        pltpu.sync_copy(zeros_hbm, tab.at[pl.ds(s * SEG_PER_SUB, SEG_PER_SUB)])
    for g in range(8):
        ones_v[pl.ds(g * 16, 16)] = jnp.ones((16,), _f32)
    plsc.subcore_barrier()

    row_lo = wid * ROWS_PER_W
    nrows = jnp.minimum(ROWS_PER_W, ROWS - row_lo)
    nchunks = nrows // KR

    def scatter_loop(vhbm, ihbm, tsum, tcnt):
        def chunk(ci, carry):
            base = row_lo + ci * KR
            pltpu.sync_copy(vhbm.at[pl.ds(base, KR)], vals_v)
            pltpu.sync_copy(ihbm.at[pl.ds(base * 128, KR * 128)], ids_v)
            handles = []
            for j in range(KR):
                idv = ids_v.at[pl.ds(j * 128, 128)]
                handles.append(pltpu.async_copy(
                    vals_v.at[j], tsum.at[idv], sem, add=True))
                handles.append(pltpu.async_copy(
                    ones_v, tcnt.at[idv], sem, add=True))
            for h in handles:
                h.wait()
            return carry
        lax.fori_loop(0, nchunks, chunk, 0)

    scatter_loop(rl_hbm, idhs_hbm, tsum_hs, tcnt_hs)
    scatter_loop(lpks_hbm, idpks_hbm, tsum_pks, tcnt_pks)

    plsc.subcore_barrier()
    for t, tab in enumerate((tsum_hs, tcnt_hs, tsum_pks, tcnt_pks)):
        sl = pl.ds(s * SEG_PER_SUB, SEG_PER_SUB)
        pltpu.sync_copy(tab.at[sl], out_hbm.at[c, t, sl])


@functools.partial(
    pl.kernel,
    out_type=jax.ShapeDtypeStruct((NW, 48), _f32),
    mesh=_mesh,
    scratch_types=[
        [pltpu.VMEM((SEG_PER_W,), _f32) for _ in range(8)],
        pltpu.VMEM((SEG_PER_W,), _f32),
        pltpu.VMEM((SEG_PER_W,), _f32),
        pltpu.VMEM((48,), _f32),
    ],
)
def _sc_finalize(tables_hbm, pit_hbm, pip_hbm, out_hbm, tb, pit_v, pip_v,
                 obuf):
    c = lax.axis_index("c")
    s = lax.axis_index("s")
    wid = c * 16 + s
    base = wid * SEG_PER_W
    sl = pl.ds(base, SEG_PER_W)
    k = 0
    for t in range(4):
        for core in range(2):
            pltpu.sync_copy(tables_hbm.at[core, t, sl], tb[k])
            k += 1
    pltpu.sync_copy(pit_hbm.at[sl], pit_v)
    pltpu.sync_copy(pip_hbm.at[sl], pip_v)

    zero = jnp.zeros((16,), _f32)

    def body(i, accs):
        a_hs, a_pks, a_pi = accs
        v = pl.ds(i * 16, 16)
        sh = tb[0][v] + tb[1][v]
        ch = jnp.maximum(tb[2][v] + tb[3][v], 1.0)
        a_hs = a_hs + sh / ch
        sp = tb[4][v] + tb[5][v]
        cp = jnp.maximum(tb[6][v] + tb[7][v], 1.0)
        a_pks = a_pks + sp / cp
        dpi = pip_v[v] - pit_v[v]
        a_pi = a_pi + dpi * dpi
        return (a_hs, a_pks, a_pi)

    a_hs, a_pks, a_pi = lax.fori_loop(0, SEG_PER_W // 16, body,
                                      (zero, zero, zero))
    obuf[pl.ds(0, 16)] = a_hs
    obuf[pl.ds(16, 16)] = a_pks
    obuf[pl.ds(32, 16)] = a_pi
    pltpu.sync_copy(obuf, out_hbm.at[wid])


def kernel(y_hs_true, y_hs_pred, y_hs_batch, y_pks_true, y_pks_pred,
           y_pks_batch, y_pI_true, y_pI_pred):
    pks_t2 = y_pks_true.reshape(ROWS, 128)
    pks_p2 = y_pks_pred.reshape(ROWS, 128)
    hs_t2 = y_hs_true.reshape(HS_ROWS, 128)
    hs_p2 = y_hs_pred.reshape(HS_ROWS, 128)
    rl_hs, l_pks = _tc_row_losses(hs_t2, hs_p2, pks_t2, pks_p2)

    idhs = y_hs_batch.astype(jnp.int32)
    idpks = y_pks_batch.astype(jnp.int32)
    zeros_hbm = jnp.zeros((SEG_PER_SUB,), _f32)

    tables = _sc_accum(rl_hs, idhs, l_pks, idpks, zeros_hbm)

    pit = jnp.pad(y_pI_true, (0, S_PAD - S))
    pip = jnp.pad(y_pI_pred, (0, S_PAD - S))
    parts = _sc_finalize(tables, pit, pip)

    tot = jnp.sum(parts, axis=0)
    multitask = jnp.stack([
        jnp.sum(tot[0:16]), jnp.sum(tot[16:32]), jnp.sum(tot[32:48])
    ]) * (1.0 / S)
    return (multitask, jnp.zeros(1), jnp.zeros(1))
